# baseline (speedup 1.0000x reference)
import functools

import jax
import jax.numpy as jnp
from jax import lax
from jax.experimental import pallas as pl
from jax.experimental.pallas import tpu as pltpu
from jax.experimental.pallas import tpu_sc as plsc

N = 16384
EMB = 128
H2 = 512
OUT = 256
NW = 32
NCHUNK = 4
CN = N // NCHUNK
CRPW = CN // NW
BLK = 2048

_sc_mesh = plsc.VectorSubcoreMesh(core_axis_name="c", subcore_axis_name="s")


@functools.partial(
    pl.kernel,
    mesh=_sc_mesh,
    out_type=(
        jax.ShapeDtypeStruct((CN, EMB), jnp.float32),
        jax.ShapeDtypeStruct((CN, EMB), jnp.float32),
    ),
    scratch_types=[
        pltpu.VMEM((CRPW,), jnp.int32),
        pltpu.VMEM((CRPW, EMB), jnp.float32),
        pltpu.SemaphoreType.DMA,
    ],
)
def _sc_gather(gtab, ktab, gidx, kidx, gout, kout, idx_v, rows_v, sem):
    wid = lax.axis_index("s") * 2 + lax.axis_index("c")
    base = wid * CRPW
    for tab, out_hbm, idx_hbm in ((gtab, gout, gidx), (ktab, kout, kidx)):
        pltpu.sync_copy(idx_hbm.at[pl.ds(base, CRPW)], idx_v)
        pltpu.async_copy(tab.at[idx_v], rows_v, sem).wait()
        pltpu.sync_copy(rows_v, out_hbm.at[pl.ds(base, CRPW)])


def _mlp_compute(g_ref, k_ref, w1_ref, w2_ref, b1_ref, b2_ref, o_ref):
    dnums = (((1,), (1,)), ((), ()))
    h = lax.dot_general(
        g_ref[...], w1_ref[:, :EMB], dnums, preferred_element_type=jnp.float32
    )
    h = h + lax.dot_general(
        k_ref[...], w1_ref[:, EMB:], dnums, preferred_element_type=jnp.float32
    )
    h = jnp.maximum(h + b1_ref[...], 0.0)
    o_ref[...] = (
        lax.dot_general(h, w2_ref[...], dnums, preferred_element_type=jnp.float32)
        + b2_ref[...]
    )


def _mlp_body_first(g_ref, k_ref, w1_ref, w2_ref, b1_ref, b2_ref, o_ref):
    _mlp_compute(g_ref, k_ref, w1_ref, w2_ref, b1_ref, b2_ref, o_ref)


def _mlp_body_alias(g_ref, k_ref, w1_ref, w2_ref, b1_ref, b2_ref, prev_ref, o_ref):
    del prev_ref
    _mlp_compute(g_ref, k_ref, w1_ref, w2_ref, b1_ref, b2_ref, o_ref)


def _mlp_chunk(c, gbuf, kbuf, w1, w2, b1, b2, out_prev):
    blocks_per_chunk = CN // BLK
    in_specs = [
        pl.BlockSpec((BLK, EMB), lambda i: (i, 0)),
        pl.BlockSpec((BLK, EMB), lambda i: (i, 0)),
        pl.BlockSpec((H2, 2 * EMB), lambda i: (0, 0)),
        pl.BlockSpec((OUT, H2), lambda i: (0, 0)),
        pl.BlockSpec((1, H2), lambda i: (0, 0)),
        pl.BlockSpec((1, OUT), lambda i: (0, 0)),
    ]
    out_spec = pl.BlockSpec(
        (BLK, OUT), lambda i, c=c: (c * blocks_per_chunk + i, 0)
    )
    args = [gbuf, kbuf, w1, w2, b1, b2]
    if out_prev is None:
        body = _mlp_body_first
        kwargs = {}
    else:
        body = _mlp_body_alias
        in_specs = in_specs + [pl.BlockSpec(memory_space=pl.ANY)]
        args = args + [out_prev]
        kwargs = {"input_output_aliases": {6: 0}}
    return pl.pallas_call(
        body,
        grid=(blocks_per_chunk,),
        in_specs=in_specs,
        out_specs=out_spec,
        out_shape=jax.ShapeDtypeStruct((N, OUT), jnp.float32),
        **kwargs,
    )(*args)


def kernel(input_tensor, genre_table, key_table, W1, b1, W2, b2):
    g_idx = input_tensor[:, 0]
    k_idx = input_tensor[:, 1]
    b1r = b1.reshape(1, H2)
    b2r = b2.reshape(1, OUT)
    embs = []
    for c in range(NCHUNK):
        embs.append(
            _sc_gather(
                genre_table,
                key_table,
                lax.dynamic_slice_in_dim(g_idx, c * CN, CN),
                lax.dynamic_slice_in_dim(k_idx, c * CN, CN),
            )
        )
    out = None
    for c in range(NCHUNK):
        gbuf, kbuf = embs[c]
        out = _mlp_chunk(c, gbuf, kbuf, W1, W2, b1r, b2r, out)
    return out


# speedup vs baseline: 1.1646x; 1.1646x over previous
import functools

import jax
import jax.numpy as jnp
from jax import lax
from jax.experimental import pallas as pl
from jax.experimental.pallas import tpu as pltpu
from jax.experimental.pallas import tpu_sc as plsc

N = 16384
EMB = 128
H2 = 512
OUT = 256
NW = 32
RPW = N // NW
IDX_W = 128
CHUNKS = RPW // IDX_W
BLK = 2048

_sc_mesh = plsc.VectorSubcoreMesh(core_axis_name="c", subcore_axis_name="s")


@functools.partial(
    pl.kernel,
    mesh=_sc_mesh,
    out_type=(
        jax.ShapeDtypeStruct((N, EMB), jnp.float32),
        jax.ShapeDtypeStruct((N, EMB), jnp.float32),
    ),
    scratch_types=[
        pltpu.VMEM((CHUNKS, IDX_W), jnp.int32),
        pltpu.VMEM((RPW, EMB), jnp.float32),
        pltpu.SemaphoreType.DMA,
    ],
)
def _sc_gather(gtab, ktab, gidx, kidx, gout, kout, idx_v, rows_v, sem):
    wid = lax.axis_index("s") * 2 + lax.axis_index("c")
    row0 = wid * RPW
    blk0 = wid * CHUNKS

    def one_table(tab, out_hbm, idx_hbm):
        pltpu.sync_copy(idx_hbm.at[pl.ds(blk0, CHUNKS)], idx_v)
        copies = []
        for j in range(CHUNKS):
            copies.append(
                pltpu.async_copy(
                    tab.at[idx_v.at[j]], rows_v.at[pl.ds(j * IDX_W, IDX_W)], sem
                )
            )
        for c in copies:
            c.wait()
        pltpu.sync_copy(rows_v, out_hbm.at[pl.ds(row0, RPW)])

    one_table(gtab, gout, gidx)
    one_table(ktab, kout, kidx)


def _mlp_body(g_ref, k_ref, w1_ref, w2_ref, b1_ref, b2_ref, o_ref):
    dnums = (((1,), (1,)), ((), ()))
    bf = jnp.bfloat16
    emb = jnp.concatenate(
        [g_ref[...].astype(bf), k_ref[...].astype(bf)], axis=1
    )
    h = lax.dot_general(
        emb, w1_ref[...].astype(bf), dnums, preferred_element_type=jnp.float32
    )
    h16 = jnp.maximum(h.astype(bf) + b1_ref[...].astype(bf), jnp.asarray(0, bf))
    o_ref[...] = (
        lax.dot_general(
            h16, w2_ref[...].astype(bf), dnums, preferred_element_type=jnp.float32
        )
        + b2_ref[...]
    )


def _mlp(gbuf, kbuf, w1, w2, b1, b2):
    return pl.pallas_call(
        _mlp_body,
        grid=(N // BLK,),
        in_specs=[
            pl.BlockSpec((BLK, EMB), lambda i: (i, 0)),
            pl.BlockSpec((BLK, EMB), lambda i: (i, 0)),
            pl.BlockSpec((H2, 2 * EMB), lambda i: (0, 0)),
            pl.BlockSpec((OUT, H2), lambda i: (0, 0)),
            pl.BlockSpec((1, H2), lambda i: (0, 0)),
            pl.BlockSpec((1, OUT), lambda i: (0, 0)),
        ],
        out_specs=pl.BlockSpec((BLK, OUT), lambda i: (i, 0)),
        out_shape=jax.ShapeDtypeStruct((N, OUT), jnp.float32),
    )(gbuf, kbuf, w1, w2, b1, b2)


def kernel(input_tensor, genre_table, key_table, W1, b1, W2, b2):
    g_idx = input_tensor[:, 0].reshape(N // IDX_W, IDX_W)
    k_idx = input_tensor[:, 1].reshape(N // IDX_W, IDX_W)
    gbuf, kbuf = _sc_gather(genre_table, key_table, g_idx, k_idx)
    return _mlp(gbuf, kbuf, W1, W2, b1.reshape(1, H2), b2.reshape(1, OUT))
